# speedup vs baseline: 1.0140x; 1.0140x over previous
import jax
import jax.numpy as jnp
from jax.experimental import pallas as pl
from jax.experimental.pallas import tpu as pltpu

N = 10000
E = 320000
HID = 128
HEADS = 8
HDIM = HID // HEADS
NUM_GCL = 2


def _mm_bias_kernel(x_ref, w_ref, b_ref, o_ref):
    o_ref[...] = jnp.dot(x_ref[...], w_ref[...], preferred_element_type=jnp.float32) + b_ref[...]


def _mm_bias(x, w, b):
    n, k = x.shape
    m = w.shape[1]
    return pl.pallas_call(
        _mm_bias_kernel,
        out_shape=jax.ShapeDtypeStruct((n, m), jnp.float32),
    )(x, w, b[None, :])


def _seg_softmax(logits, seg, n):
    m = jax.ops.segment_max(logits, seg, num_segments=n)
    m = jnp.where(jnp.isfinite(m), m, 0.0)
    e = jnp.exp(logits - m[seg])
    s = jax.ops.segment_sum(e, seg, num_segments=n)
    return e / (s[seg] + 1e-16)


def kernel(x, W_enc, b_enc, att_src_e, att_dst_e, W_gate, b_gate, W_gcl, W_q, att_src_d, att_dst_d, W_out, b_out, edge_index, mask):
    src = edge_index[0]
    dst = edge_index[1]
    h = _mm_bias(x, W_enc, b_enc)
    logit = jax.nn.leaky_relu(h[src] @ att_src_e + h[dst] @ att_dst_e, negative_slope=0.2)
    alpha = _seg_softmax(logit, dst, N)
    agg = jax.ops.segment_sum(alpha[:, None] * h[src], dst, num_segments=N)
    gate = jax.nn.sigmoid(_mm_bias(h, W_gate, b_gate))
    h = jax.nn.relu(gate * agg + (1.0 - gate) * h)
    deg = jax.ops.segment_sum(jnp.ones((E,), jnp.float32), dst, num_segments=N) + 1.0
    inv_sqrt = 1.0 / jnp.sqrt(deg)
    norm = inv_sqrt[src] * inv_sqrt[dst]
    for l in range(NUM_GCL):
        msg = h @ W_gcl[l]
        agg = jax.ops.segment_sum(norm[:, None] * msg[src], dst, num_segments=N)
        h = jax.nn.relu(h + agg)
    q = (h @ W_q).reshape(N, HEADS, HDIM)
    lg = jax.nn.leaky_relu(jnp.sum(q[src] * att_src_d[None, :, :], axis=-1) + jnp.sum(q[dst] * att_dst_d[None, :, :], axis=-1), negative_slope=0.2)
    al = _seg_softmax(lg, dst, N)
    aggd = jax.ops.segment_sum(al[:, :, None] * q[src], dst, num_segments=N).reshape(N, HID)
    out = _mm_bias(aggd, W_out, b_out)
    return jnp.take(out, mask, axis=0)


# speedup vs baseline: 30.2981x; 29.8794x over previous
import functools

import jax
import jax.numpy as jnp
from jax import lax
from jax.experimental import pallas as pl
from jax.experimental.pallas import tpu as pltpu
from jax.experimental.pallas import tpu_sc as plsc

F32 = jnp.float32
I32 = jnp.int32

NN = 10000
NPAD = 10112
EE = 320000
NW = 32
EPT = EE // NW
NB = 79
EPAD = NB * 128
RPT = NPAD // 16
HID = 128
OUT = 64
MM = 5000
MPAD = 5120

_mesh = plsc.VectorSubcoreMesh(
    core_axis_name="c", subcore_axis_name="s", num_cores=2, num_subcores=16)


def _wid():
    return lax.axis_index("s") * 2 + lax.axis_index("c")


def _enc_scalar_body(src_hbm, dst_hbm, t0_hbm, ev_out, parts_out,
                     src_v, dst_v, t0_v, ev_v, dacc, gacc):
    wid = _wid()
    pltpu.sync_copy(src_hbm.at[wid], src_v)
    pltpu.sync_copy(dst_hbm.at[wid], dst_v)
    pltpu.sync_copy(t0_hbm, t0_v)
    z16 = jnp.zeros((16,), F32)

    def zr(g, _):
        off = pl.ds(pl.multiple_of(g * 16, 16), 16)
        dacc[off] = z16
        gacc[off] = z16
        return 0
    lax.fori_loop(0, NPAD // 16, zr, 0)

    col1 = jnp.ones((16,), I32)
    ones16 = jnp.ones((16,), F32)

    def ed(g, _):
        off = pl.ds(pl.multiple_of(g * 16, 16), 16)
        s16 = src_v[off]
        d16 = dst_v[off]
        vs = plsc.load_gather(t0_v, [s16 * 2])
        vd = plsc.load_gather(t0_v, [d16 * 2 + col1])
        l = vs + vd
        l = jnp.where(l >= 0, l, l * 0.2)
        ev = jnp.exp(l)
        ev_v[off] = ev
        plsc.addupdate_scatter(dacc, [d16], ev)
        plsc.addupdate_scatter(gacc, [d16], ones16)
        return 0
    lax.fori_loop(0, EPAD // 16, ed, 0)

    pltpu.sync_copy(ev_v, ev_out.at[wid])
    pltpu.sync_copy(dacc, parts_out.at[0, wid])
    pltpu.sync_copy(gacc, parts_out.at[1, wid])


_enc_scalars = pl.kernel(
    _enc_scalar_body,
    out_type=(jax.ShapeDtypeStruct((NW, EPAD), F32),
              jax.ShapeDtypeStruct((2, NW, NPAD), F32)),
    mesh=_mesh,
    compiler_params=pltpu.CompilerParams(needs_layout_passes=False),
    scratch_types=[
        pltpu.VMEM((EPAD,), I32),
        pltpu.VMEM((EPAD,), I32),
        pltpu.VMEM((NPAD * 2,), F32),
        pltpu.VMEM((EPAD,), F32),
        pltpu.VMEM((NPAD,), F32),
        pltpu.VMEM((NPAD,), F32),
    ],
)


def _spmm_body(mode, *refs):
    if mode == "enc":
        (h_hbm, src_hbm, dst3_hbm, ev_hbm, z_hbm, out_hbm,
         src_v, dst3_v, ev_v, rows_v, acc_sh, sem) = refs
    elif mode == "dec":
        (h_hbm, src_hbm, dst3_hbm, ev_hbm, z_hbm, out_hbm,
         src_v, dst3_v, ev_v, rows_v, acc_sh, sem) = refs
    else:
        (h_hbm, src_hbm, dst3_hbm, z_hbm, out_hbm,
         src_v, dst3_v, rows_v, acc_sh, sem) = refs
    c = lax.axis_index("c")
    s = lax.axis_index("s")
    wid = s * 2 + c
    row_sl = pl.ds(pl.multiple_of(s * RPT, RPT), RPT)
    pltpu.sync_copy(z_hbm.at[row_sl], acc_sh.at[row_sl])
    pltpu.sync_copy(src_hbm.at[wid], src_v)
    pltpu.sync_copy(dst3_hbm.at[wid], dst3_v)
    if mode == "enc":
        pltpu.sync_copy(ev_hbm.at[wid], ev_v)
    plsc.subcore_barrier()

    def batch(j, _):
        off = pl.ds(pl.multiple_of(j * 128, 128), 128)
        if mode == "dec":
            pltpu.sync_copy(
                ev_hbm.at[wid, pl.ds(pl.multiple_of(j * 1024, 1024), 1024)],
                ev_v)
        pltpu.async_copy(h_hbm.at[src_v.at[off]], rows_v, sem).wait()
        if mode == "enc":
            def grp(g, _):
                av16 = ev_v[pl.ds(pl.multiple_of(j * 128 + g * 16, 16), 16)]
                for i in range(16):
                    e = g * 16 + i
                    av = jnp.full((16,), av16[i], F32)
                    for k in range(8):
                        sl = pl.ds(k * 16, 16)
                        rows_v[e, sl] = rows_v[e, sl] * av
                return 0
            lax.fori_loop(0, 8, grp, 0)
        elif mode == "dec":
            def grp(g, _):
                for m in range(8):
                    va = ev_v[pl.ds(pl.multiple_of(g * 128 + m * 16, 16), 16)]
                    e0 = g * 16 + 2 * m
                    for k in range(8):
                        sl = pl.ds(k * 16, 16)
                        rows_v[e0, sl] = rows_v[e0, sl] * jnp.full((16,), va[k], F32)
                        rows_v[e0 + 1, sl] = (
                            rows_v[e0 + 1, sl] * jnp.full((16,), va[8 + k], F32))
                return 0
            lax.fori_loop(0, 8, grp, 0)
        pltpu.sync_copy(rows_v, acc_sh.at[dst3_v.at[j]], add=True)
        return 0
    lax.fori_loop(0, NB, batch, 0)

    plsc.subcore_barrier()
    pltpu.sync_copy(acc_sh.at[row_sl], out_hbm.at[c, row_sl])


def _make_spmm(mode):
    scratch = [pltpu.VMEM((EPAD,), I32), pltpu.VMEM((NB, 128), I32)]
    if mode == "enc":
        scratch.append(pltpu.VMEM((EPAD,), F32))
    elif mode == "dec":
        scratch.append(pltpu.VMEM((1024,), F32))
    scratch += [
        pltpu.VMEM((128, 128), F32),
        pltpu.VMEM_SHARED((NPAD, 128), F32),
        pltpu.SemaphoreType.DMA,
    ]
    return pl.kernel(
        functools.partial(_spmm_body, mode),
        out_type=jax.ShapeDtypeStruct((2, NPAD, 128), F32),
        mesh=_mesh,
        compiler_params=pltpu.CompilerParams(needs_layout_passes=False),
        scratch_types=scratch,
    )


_spmm_enc = _make_spmm("enc")
_spmm_plain = _make_spmm("plain")
_spmm_dec = _make_spmm("dec")


def _dec_scalar_body(src_hbm, dst_hbm, t3_hbm, ev_out, parts_out,
                     src_v, dst_v, t3_v, ev_v, da, db):
    wid = _wid()
    pltpu.sync_copy(src_hbm.at[wid], src_v)
    pltpu.sync_copy(dst_hbm.at[wid], dst_v)
    iota = lax.iota(I32, 16)
    half = iota >> 1
    par = iota & 1
    z16 = jnp.zeros((16,), F32)

    for p in range(4):
        pltpu.sync_copy(t3_hbm.at[p], t3_v)

        def zr(g, _):
            off = pl.ds(pl.multiple_of(g * 16, 16), 16)
            da[off] = z16
            db[off] = z16
            return 0
        lax.fori_loop(0, NPAD // 16, zr, 0)

        def ed(g, _):
            e_idx = g * 8 + half
            s_r = plsc.load_gather(src_v, [e_idx])
            d_r = plsc.load_gather(dst_v, [e_idx])
            vs = plsc.load_gather(t3_v, [s_r * 4 + par])
            vd = plsc.load_gather(t3_v, [d_r * 4 + par + 2])
            l = vs + vd
            l = jnp.where(l >= 0, l, l * 0.2)
            ev = jnp.exp(l)
            ev_v[pl.ds(pl.multiple_of(g * 16, 16), 16)] = ev
            plsc.addupdate_scatter(da, [d_r], ev, mask=par == 0)
            plsc.addupdate_scatter(db, [d_r], ev, mask=par == 1)
            return 0
        lax.fori_loop(0, 2 * EPAD // 16, ed, 0)

        pltpu.sync_copy(ev_v, ev_out.at[p, wid])
        pltpu.sync_copy(da, parts_out.at[2 * p, wid])
        pltpu.sync_copy(db, parts_out.at[2 * p + 1, wid])


_dec_scalars = pl.kernel(
    _dec_scalar_body,
    out_type=(jax.ShapeDtypeStruct((4, NW, 2 * EPAD), F32),
              jax.ShapeDtypeStruct((8, NW, NPAD), F32)),
    mesh=_mesh,
    compiler_params=pltpu.CompilerParams(needs_layout_passes=False),
    scratch_types=[
        pltpu.VMEM((EPAD,), I32),
        pltpu.VMEM((EPAD,), I32),
        pltpu.VMEM((NPAD * 4,), F32),
        pltpu.VMEM((2 * EPAD,), F32),
        pltpu.VMEM((NPAD,), F32),
        pltpu.VMEM((NPAD,), F32),
    ],
)


def _mask_gather_body(tab_hbm, mask_hbm, out_hbm, idx_v, rows_v, sem):
    wid = _wid()
    pltpu.sync_copy(mask_hbm.at[wid], idx_v)
    pltpu.async_copy(tab_hbm.at[idx_v.at[pl.ds(0, 128)]],
                     rows_v.at[pl.ds(0, 128)], sem).wait()
    pltpu.async_copy(tab_hbm.at[idx_v.at[pl.ds(128, 32)]],
                     rows_v.at[pl.ds(128, 32)], sem).wait()
    pltpu.sync_copy(rows_v, out_hbm.at[pl.ds(wid * 160, 160)])


_mask_gather = pl.kernel(
    _mask_gather_body,
    out_type=jax.ShapeDtypeStruct((MPAD, 128), F32),
    mesh=_mesh,
    compiler_params=pltpu.CompilerParams(needs_layout_passes=False),
    scratch_types=[
        pltpu.VMEM((160,), I32),
        pltpu.VMEM((160, 128), F32),
        pltpu.SemaphoreType.DMA,
    ],
)


def _tca_body(x, we, be, wg, bg, ap, h0_o, gate_o, t0_o):
    h = jnp.dot(x[...], we[...], preferred_element_type=F32) + be[...]
    h0_o[...] = h
    gate_o[...] = jax.nn.sigmoid(
        jnp.dot(h, wg[...], preferred_element_type=F32) + bg[...])
    t0_o[...] = jnp.dot(h, ap[...], preferred_element_type=F32)


def _tca(x, we, be, wg, bg, ap):
    return pl.pallas_call(
        _tca_body,
        out_shape=(jax.ShapeDtypeStruct((NN, HID), F32),
                   jax.ShapeDtypeStruct((NN, HID), F32),
                   jax.ShapeDtypeStruct((NN, 2), F32)),
    )(x, we, be, wg, bg, ap)


def _tcc_body(aggp, parts, h0, gate, wg0, h1_o, msg1_o, rsq_o):
    denom = jnp.sum(parts[0], axis=0)[:NN]
    deg = jnp.sum(parts[1], axis=0)[:NN] + 1.0
    rden = (1.0 / (denom + 1e-16))[:, None]
    rsq = lax.rsqrt(deg)[:, None]
    agg = (aggp[0, :NN, :] + aggp[1, :NN, :]) * rden
    g = gate[...]
    h1 = jnp.maximum(g * agg + (1.0 - g) * h0[...], 0.0)
    h1_o[...] = h1
    msg1_o[...] = jnp.dot(h1, wg0[...], preferred_element_type=F32) * rsq
    rsq_o[...] = rsq


def _tcc(aggp, parts, h0, gate, wg0):
    return pl.pallas_call(
        _tcc_body,
        out_shape=(jax.ShapeDtypeStruct((NN, HID), F32),
                   jax.ShapeDtypeStruct((NN, HID), F32),
                   jax.ShapeDtypeStruct((NN, 1), F32)),
    )(aggp, parts, h0, gate, wg0)


def _tcd_body(aggp, h1, rsq, wg1, h2_o, msg2_o):
    agg = (aggp[0, :NN, :] + aggp[1, :NN, :]) * rsq[...]
    h2 = jnp.maximum(h1[...] + agg, 0.0)
    h2_o[...] = h2
    msg2_o[...] = jnp.dot(h2, wg1[...], preferred_element_type=F32) * rsq[...]


def _tcd(aggp, h1, rsq, wg1):
    return pl.pallas_call(
        _tcd_body,
        out_shape=(jax.ShapeDtypeStruct((NN, HID), F32),
                   jax.ShapeDtypeStruct((NN, HID), F32)),
    )(aggp, h1, rsq, wg1)


def _tce_body(aggp, h2, rsq, wq, asm, adm, q_o, ss_o, sd_o):
    agg = (aggp[0, :NN, :] + aggp[1, :NN, :]) * rsq[...]
    h3 = jnp.maximum(h2[...] + agg, 0.0)
    q = jnp.dot(h3, wq[...], preferred_element_type=F32)
    q_o[...] = q
    ss_o[...] = jnp.dot(q, asm[...], preferred_element_type=F32)
    sd_o[...] = jnp.dot(q, adm[...], preferred_element_type=F32)


def _tce(aggp, h2, rsq, wq, asm, adm):
    return pl.pallas_call(
        _tce_body,
        out_shape=(jax.ShapeDtypeStruct((NN, HID), F32),
                   jax.ShapeDtypeStruct((NN, 8), F32),
                   jax.ShapeDtypeStruct((NN, 8), F32)),
    )(aggp, h2, rsq, wq, asm, adm)


def _tcf_body(aggp, parts, wout, bout, out_o):
    aggd = aggp[0, :NN, :] + aggp[1, :NN, :]
    blocks = []
    for k in range(8):
        dn = jnp.sum(parts[k], axis=0)[:NN]
        r = (1.0 / (dn + 1e-16))[:, None]
        blocks.append(aggd[:, k * 16:(k + 1) * 16] * r)
    scaled = jnp.concatenate(blocks, axis=1)
    out_o[...] = jnp.dot(scaled, wout[...], preferred_element_type=F32) + bout[...]


def _tcf(aggp, parts, wout, bout):
    return pl.pallas_call(
        _tcf_body,
        out_shape=jax.ShapeDtypeStruct((NN, 128), F32),
    )(aggp, parts, wout, bout)


def kernel(x, W_enc, b_enc, att_src_e, att_dst_e, W_gate, b_gate, W_gcl,
           W_q, att_src_d, att_dst_d, W_out, b_out, edge_index, mask):
    src = edge_index[0]
    dst = edge_index[1]
    src_p = jnp.pad(src.reshape(NW, EPT), ((0, 0), (0, EPAD - EPT)))
    dst_p = jnp.pad(dst.reshape(NW, EPT), ((0, 0), (0, EPAD - EPT)),
                    constant_values=NN)
    dst3 = dst_p.reshape(NW, NB, 128)
    mask_p = jnp.pad(mask, (0, MPAD - MM)).reshape(NW, 160)
    zrows = jnp.zeros((NPAD, 128), F32)

    ap = jnp.stack([att_src_e, att_dst_e], axis=1)
    rows = jnp.arange(128)
    asm = jnp.zeros((128, 8), F32).at[rows, rows // 16].set(att_src_d.reshape(-1))
    adm = jnp.zeros((128, 8), F32).at[rows, rows // 16].set(att_dst_d.reshape(-1))

    h0, gate, t0 = _tca(x, W_enc, b_enc[None], W_gate, b_gate[None], ap)
    t0p = jnp.concatenate([t0, jnp.zeros((NPAD - NN, 2), F32)], axis=0).reshape(-1)

    ev, parts = _enc_scalars(src_p, dst_p, t0p)
    aggp = _spmm_enc(h0, src_p, dst3, ev, zrows)
    h1, msg1, rsq = _tcc(aggp, parts, h0, gate, W_gcl[0])

    aggp1 = _spmm_plain(msg1, src_p, dst3, zrows)
    h2, msg2 = _tcd(aggp1, h1, rsq, W_gcl[1])

    aggp2 = _spmm_plain(msg2, src_p, dst3, zrows)
    q, ss, sd = _tce(aggp2, h2, rsq, W_q, asm, adm)

    ssp = jnp.concatenate([ss, jnp.zeros((NPAD - NN, 8), F32)], axis=0)
    sdp = jnp.concatenate([sd, jnp.zeros((NPAD - NN, 8), F32)], axis=0)
    t3r = jnp.stack([
        jnp.concatenate([ssp[:, 2 * p:2 * p + 2], sdp[:, 2 * p:2 * p + 2]],
                        axis=1).reshape(-1) for p in range(4)])

    evd, partsd = _dec_scalars(src_p, dst_p, t3r)
    evr = evd.reshape(4, NW, EPAD, 2).transpose(1, 2, 0, 3).reshape(NW, EPAD * 8)
    aggpd = _spmm_dec(q, src_p, dst3, evr, zrows)
    woutp = jnp.concatenate([W_out, jnp.zeros((HID, 128 - OUT), F32)], axis=1)
    boutp = jnp.concatenate([b_out, jnp.zeros((128 - OUT,), F32)])[None]
    outf = _tcf(aggpd, partsd, woutp, boutp)

    g = _mask_gather(outf, mask_p)
    return g[:MM, :OUT]
